# baseline (speedup 1.0000x reference)
import functools

import jax
import jax.numpy as jnp
from jax import lax
from jax.experimental import pallas as pl
from jax.experimental.pallas import tpu as pltpu
from jax.experimental.pallas import tpu_sc as plsc


def kernel(frames):
    B, C, T, H, W = frames.shape
    S = T // 4
    ROWS = B * C * S
    D = H * W
    L = 16
    NG = ROWS // L
    CPR = 14
    CH = D // CPR

    flat = frames.reshape(B * C * T * CPR, CH)
    mesh = plsc.VectorSubcoreMesh(core_axis_name="c", subcore_axis_name="s")

    @functools.partial(
        pl.kernel,
        out_type=jax.ShapeDtypeStruct((ROWS, D), frames.dtype),
        mesh=mesh,
        scratch_types=[
            pltpu.VMEM((L,), jnp.int32),
            pltpu.VMEM((L,), jnp.int32),
            pltpu.VMEM((L, CH), frames.dtype),
            pltpu.VMEM((L, CH), frames.dtype),
            pltpu.SemaphoreType.DMA,
            pltpu.SemaphoreType.DMA,
            pltpu.SemaphoreType.DMA,
            pltpu.SemaphoreType.DMA,
        ],
    )
    def pack_slow(src_hbm, out_hbm, idx0, idx1, buf0, buf1, si0, si1, so0, so1):
        wid = lax.axis_index("s") * 2 + lax.axis_index("c")
        idx = (idx0, idx1)
        buf = (buf0, buf1)
        sin = (si0, si1)
        sout = (so0, so1)

        @pl.when(wid < NG)
        def _():
            base = wid * L
            lane = lax.iota(jnp.int32, L)
            tp = lane & (S - 1)
            bc = wid * (L // S) + (lane >> 3)
            src_row = bc * T + ((tp * 567) >> 7)

            def gather(c):
                return pltpu.make_async_copy(
                    src_hbm.at[idx[c % 2]], buf[c % 2], sin[c % 2])

            def start_gather(c):
                idx[c % 2][...] = src_row * CPR + c
                gather(c).start()

            def scatter(c):
                return pltpu.make_async_copy(
                    buf[c % 2],
                    out_hbm.at[pl.ds(base, L), pl.ds(c * CH, CH)],
                    sout[c % 2])

            start_gather(0)
            for c in range(CPR):
                if c + 1 < CPR:
                    if c >= 1:
                        scatter(c - 1).wait()
                    start_gather(c + 1)
                gather(c).wait()
                scatter(c).start()
            scatter(CPR - 2).wait()
            scatter(CPR - 1).wait()

    slow = pack_slow(flat).reshape(B, C, S, H, W)
    return (slow, frames)


# speedup vs baseline: 1.6836x; 1.6836x over previous
import jax
import jax.numpy as jnp
from jax.experimental import pallas as pl
from jax.experimental.pallas import tpu as pltpu


def kernel(frames):
    B, C, T, H, W = frames.shape
    S = T // 4
    BC = B * C
    idx = jnp.asarray([(t * (T - 1)) // (S - 1) for t in range(S)],
                      dtype=jnp.int32)
    src = frames.reshape(BC, T, H, W)

    def body(s_ref, in_ref, out_ref):
        out_ref[...] = in_ref[...]

    grid_spec = pltpu.PrefetchScalarGridSpec(
        num_scalar_prefetch=1,
        grid=(BC, S),
        in_specs=[pl.BlockSpec((1, 1, H, W), lambda bc, t, s: (bc, s[t], 0, 0))],
        out_specs=pl.BlockSpec((1, 1, H, W), lambda bc, t, s: (bc, t, 0, 0)),
    )
    slow = pl.pallas_call(
        body,
        grid_spec=grid_spec,
        out_shape=jax.ShapeDtypeStruct((BC, S, H, W), frames.dtype),
    )(idx, src).reshape(B, C, S, H, W)
    return (slow, frames)
